# baseline (speedup 1.0000x reference)
import functools

import jax
import jax.numpy as jnp
import numpy as np
from jax import lax
from jax.experimental import pallas as pl
from jax.experimental.pallas import tpu as pltpu
from jax.experimental.pallas import tpu_sc as plsc

N = 10000
E = 320000
D_IN = 128
D_H = 256
L = 3
DHALF = 128

NSUB = 16
C = 128
CPT = 160
EPAD = NSUB * CPT * C
NPAD = 10240
RPS = NPAD // NSUB

MB = 8
NBLK = CPT // MB

BN = 400
GRID_N = N // BN

_SIGMA = np.empty(DHALF, np.int32)
for _g in range(4):
    for _j in range(16):
        _SIGMA[32 * _g + 2 * _j] = 32 * _g + _j
        _SIGMA[32 * _g + 2 * _j + 1] = 32 * _g + 16 + _j
_PERM = np.concatenate([_SIGMA, DHALF + _SIGMA])


def _mm_body(h_ref, w_ref, o_ref):
    o_ref[...] = jnp.dot(h_ref[...], w_ref[...], preferred_element_type=jnp.float32)


def _mm_first(h, wp):
    return pl.pallas_call(
        _mm_body,
        out_shape=jax.ShapeDtypeStruct((2 * N, DHALF), jnp.float32),
        in_specs=[
            pl.BlockSpec((BN, D_H), lambda c, i: (i, 0)),
            pl.BlockSpec((D_H, DHALF), lambda c, i: (0, c)),
        ],
        out_specs=pl.BlockSpec((BN, DHALF), lambda c, i: (c * GRID_N + i, 0)),
        grid=(2, GRID_N),
    )(h, wp)


def _gru_core(a0_ref, a1_ref, h_ref, wi_ref, wh_ref, bi_ref, bh_ref):
    agg = jnp.concatenate([a0_ref[...], a1_ref[...]], axis=1)
    h = h_ref[...]
    gi = jnp.dot(agg, wi_ref[...], preferred_element_type=jnp.float32) + bi_ref[...]
    gh = jnp.dot(h, wh_ref[...], preferred_element_type=jnp.float32) + bh_ref[...]
    r = jax.nn.sigmoid(gi[:, :D_H] + gh[:, :D_H])
    z = jax.nn.sigmoid(gi[:, D_H:2 * D_H] + gh[:, D_H:2 * D_H])
    nn = jnp.tanh(gi[:, 2 * D_H:] + r * gh[:, 2 * D_H:])
    return (1.0 - z) * nn + z * h


def _gru_mm_body(a0_ref, a1_ref, h_ref, wi_ref, wh_ref, bi_ref, bh_ref, wn_ref,
                 h_out_ref, m_out_ref):
    h_new = _gru_core(a0_ref, a1_ref, h_ref, wi_ref, wh_ref, bi_ref, bh_ref)
    h_out_ref[...] = h_new
    m = jnp.dot(h_new, wn_ref[...], preferred_element_type=jnp.float32)
    m_out_ref[0] = m[:, :DHALF]
    m_out_ref[1] = m[:, DHALF:]


def _gru_mm(agg_flat, h, wiT, whT, bi, bh, wnp):
    h_new, m_split = pl.pallas_call(
        _gru_mm_body,
        out_shape=(
            jax.ShapeDtypeStruct((N, D_H), jnp.float32),
            jax.ShapeDtypeStruct((2, N, DHALF), jnp.float32),
        ),
        in_specs=[
            pl.BlockSpec((BN, DHALF), lambda i: (i, 0)),
            pl.BlockSpec((BN, DHALF), lambda i: (GRID_N + i, 0)),
            pl.BlockSpec((BN, D_H), lambda i: (i, 0)),
            pl.BlockSpec((D_H, 3 * D_H), lambda i: (0, 0)),
            pl.BlockSpec((D_H, 3 * D_H), lambda i: (0, 0)),
            pl.BlockSpec((1, 3 * D_H), lambda i: (0, 0)),
            pl.BlockSpec((1, 3 * D_H), lambda i: (0, 0)),
            pl.BlockSpec((D_H, D_H), lambda i: (0, 0)),
        ],
        out_specs=(
            pl.BlockSpec((BN, D_H), lambda i: (i, 0)),
            pl.BlockSpec((2, BN, DHALF), lambda i: (0, i, 0)),
        ),
        grid=(GRID_N,),
    )(agg_flat, agg_flat, h, wiT, whT, bi, bh, wnp)
    return h_new, m_split.reshape(2 * N, DHALF)


def _gru_final_body(a0_ref, a1_ref, h_ref, wi_ref, wh_ref, bi_ref, bh_ref,
                    fw_ref, fb_ref, o_ref):
    h_new = _gru_core(a0_ref, a1_ref, h_ref, wi_ref, wh_ref, bi_ref, bh_ref)
    h_new = jnp.maximum(h_new, 0.0)
    o_ref[...] = jnp.dot(h_new, fw_ref[...], preferred_element_type=jnp.float32) + fb_ref[0, 0]


def _gru_final(agg_flat, h, wiT, whT, bi, bh, fwT, fb):
    return pl.pallas_call(
        _gru_final_body,
        out_shape=jax.ShapeDtypeStruct((N, 1), jnp.float32),
        in_specs=[
            pl.BlockSpec((BN, DHALF), lambda i: (i, 0)),
            pl.BlockSpec((BN, DHALF), lambda i: (GRID_N + i, 0)),
            pl.BlockSpec((BN, D_H), lambda i: (i, 0)),
            pl.BlockSpec((D_H, 3 * D_H), lambda i: (0, 0)),
            pl.BlockSpec((D_H, 3 * D_H), lambda i: (0, 0)),
            pl.BlockSpec((1, 3 * D_H), lambda i: (0, 0)),
            pl.BlockSpec((1, 3 * D_H), lambda i: (0, 0)),
            pl.BlockSpec((D_H, 1), lambda i: (0, 0)),
            pl.BlockSpec((1, 1), lambda i: (0, 0), memory_space=pltpu.SMEM),
        ],
        out_specs=pl.BlockSpec((BN, 1), lambda i: (i, 0)),
        grid=(GRID_N,),
    )(agg_flat, agg_flat, h, wiT, whT, bi, bh, fwT, fb)


_HIMASK = jnp.int32(-65536)


def _sc_agg_body(m_hbm, src_hbm, dst_hbm, attr_hbm, zeros_hbm, out_hbm,
                 srcb, dstb, attrb, gbuf0, gbuf1, aggsh, sem0, sem1):
    c = lax.axis_index("c")
    s = lax.axis_index("s")
    pltpu.sync_copy(zeros_hbm, aggsh.at[pl.ds(s * RPS, RPS)])
    plsc.subcore_barrier()

    def scale_and_scatter(gbuf, k):
        def group(g, carry3):
            av = attrb[k, pl.ds(g * 16, 16)]
            e0 = g * 16
            for el in range(16):
                sval = av[el]
                e = e0 + el
                for v in range(8):
                    gbuf[e, pl.ds(v * 16, 16)] = gbuf[e, pl.ds(v * 16, 16)] * sval
            return carry3

        lax.fori_loop(0, C // 16, group, 0)
        pltpu.sync_copy(gbuf, aggsh.at[dstb.at[k]], add=True)

    def block(b, carry):
        row0 = s * CPT + b * MB
        pltpu.sync_copy(src_hbm.at[pl.ds((c * NSUB + s) * CPT + b * MB, MB)], srcb)
        pltpu.sync_copy(dst_hbm.at[pl.ds(row0, MB)], dstb)
        pltpu.sync_copy(attr_hbm.at[pl.ds(row0, MB)], attrb)
        pltpu.make_async_copy(m_hbm.at[srcb.at[0]], gbuf0, sem0).start()

        def pair(k2, carry2):
            k = 2 * k2
            pltpu.make_async_copy(m_hbm.at[srcb.at[k + 1]], gbuf1, sem1).start()
            pltpu.make_async_copy(m_hbm.at[pl.ds(0, C)], gbuf0, sem0).wait()
            scale_and_scatter(gbuf0, k)

            @pl.when(k2 < MB // 2 - 1)
            def _():
                pltpu.make_async_copy(m_hbm.at[srcb.at[k + 2]], gbuf0, sem0).start()

            pltpu.make_async_copy(m_hbm.at[pl.ds(0, C)], gbuf1, sem1).wait()
            scale_and_scatter(gbuf1, k + 1)
            return carry2

        lax.fori_loop(0, MB // 2, pair, 0)
        return carry

    lax.fori_loop(0, NBLK, block, 0)
    plsc.subcore_barrier()
    pltpu.sync_copy(aggsh.at[pl.ds(s * RPS, RPS)],
                    out_hbm.at[pl.ds(c * NPAD + s * RPS, RPS)])


@functools.cache
def _sc_agg():
    return pl.kernel(
        _sc_agg_body,
        out_type=jax.ShapeDtypeStruct((2 * NPAD, DHALF), jnp.float32),
        mesh=plsc.VectorSubcoreMesh(core_axis_name="c", subcore_axis_name="s",
                                    num_cores=2, num_subcores=NSUB),
        scratch_types=[
            pltpu.VMEM((MB, C), jnp.int32),
            pltpu.VMEM((MB, C), jnp.int32),
            pltpu.VMEM((MB, C), jnp.float32),
            pltpu.VMEM((C, DHALF), jnp.float32),
            pltpu.VMEM((C, DHALF), jnp.float32),
            pltpu.VMEM_SHARED((NPAD, DHALF), jnp.float32),
            pltpu.SemaphoreType.DMA,
            pltpu.SemaphoreType.DMA,
        ],
    )


def kernel(x, edge_index, edge_attr, W, w_ih, w_hh, b_ih, b_hh, fc_w, fc_b):
    h = jnp.concatenate([x, jnp.zeros((N, D_H - D_IN), dtype=x.dtype)], axis=1)
    src = edge_index[0].astype(jnp.int32)
    dst = edge_index[1].astype(jnp.int32)
    attr = edge_attr.astype(jnp.float32)
    pad = EPAD - E
    src = jnp.concatenate([src, jnp.zeros((pad,), jnp.int32)])
    dst2 = jnp.concatenate([dst, jnp.zeros((pad,), jnp.int32)]).reshape(NSUB * CPT, C)
    attr2 = jnp.concatenate([attr, jnp.zeros((pad,), jnp.float32)]).reshape(NSUB * CPT, C)
    src2 = jnp.concatenate([src, src + N]).reshape(2 * NSUB * CPT, C)
    zeros = jnp.zeros((RPS, DHALF), jnp.float32)

    wiT = w_ih.T
    whT = w_hh.T
    bi = b_ih.reshape(1, 3 * D_H)
    bh = b_hh.reshape(1, 3 * D_H)
    fwT = fc_w.T
    fb = fc_b.reshape(1, 1)

    m_flat = _mm_first(h, W[0])
    for i in range(L):
        agg_pad = _sc_agg()(m_flat, src2, dst2, attr2, zeros)
        agg_flat = jnp.concatenate([agg_pad[:N], agg_pad[NPAD:NPAD + N]], axis=0)
        if i < L - 1:
            h, m_flat = _gru_mm(agg_flat, h, wiT, whT, bi, bh, W[i + 1])
        else:
            out = _gru_final(agg_flat, h, wiT, whT, bi, bh, fwT, fb)
    return out


# speedup vs baseline: 1.0189x; 1.0189x over previous
import functools

import jax
import jax.numpy as jnp
import numpy as np
from jax import lax
from jax.experimental import pallas as pl
from jax.experimental.pallas import tpu as pltpu
from jax.experimental.pallas import tpu_sc as plsc

N = 10000
E = 320000
D_IN = 128
D_H = 256
L = 3
DHALF = 128

NSUB = 16
C = 128
CPT = 160
EPAD = NSUB * CPT * C
NPAD = 10240
RPS = NPAD // NSUB

MB = 8
NBLK = CPT // MB

BN = 1000
GRID_N = N // BN

_SIGMA = np.empty(DHALF, np.int32)
for _g in range(4):
    for _j in range(16):
        _SIGMA[32 * _g + 2 * _j] = 32 * _g + _j
        _SIGMA[32 * _g + 2 * _j + 1] = 32 * _g + 16 + _j
_PERM = np.concatenate([_SIGMA, DHALF + _SIGMA])


def _mm_body(h_ref, w_ref, o_ref):
    o_ref[...] = jnp.dot(h_ref[...], w_ref[...], preferred_element_type=jnp.float32)


def _mm_first(h, wp):
    return pl.pallas_call(
        _mm_body,
        out_shape=jax.ShapeDtypeStruct((2 * N, DHALF), jnp.float32),
        in_specs=[
            pl.BlockSpec((BN, D_H), lambda c, i: (i, 0)),
            pl.BlockSpec((D_H, DHALF), lambda c, i: (0, c)),
        ],
        out_specs=pl.BlockSpec((BN, DHALF), lambda c, i: (c * GRID_N + i, 0)),
        grid=(2, GRID_N),
    )(h, wp)


def _gru_core(a0_ref, a1_ref, h_ref, wi_ref, wh_ref, bi_ref, bh_ref):
    agg = jnp.concatenate([a0_ref[...], a1_ref[...]], axis=1)
    h = h_ref[...]
    gi = jnp.dot(agg, wi_ref[...], preferred_element_type=jnp.float32) + bi_ref[...]
    gh = jnp.dot(h, wh_ref[...], preferred_element_type=jnp.float32) + bh_ref[...]
    r = jax.nn.sigmoid(gi[:, :D_H] + gh[:, :D_H])
    z = jax.nn.sigmoid(gi[:, D_H:2 * D_H] + gh[:, D_H:2 * D_H])
    nn = jnp.tanh(gi[:, 2 * D_H:] + r * gh[:, 2 * D_H:])
    return (1.0 - z) * nn + z * h


def _gru_mm_body(a0_ref, a1_ref, h_ref, wi_ref, wh_ref, bi_ref, bh_ref, wn_ref,
                 h_out_ref, m_out_ref):
    h_new = _gru_core(a0_ref, a1_ref, h_ref, wi_ref, wh_ref, bi_ref, bh_ref)
    h_out_ref[...] = h_new
    m = jnp.dot(h_new, wn_ref[...], preferred_element_type=jnp.float32)
    m_out_ref[0] = m[:, :DHALF]
    m_out_ref[1] = m[:, DHALF:]


def _gru_mm(agg_flat, h, wiT, whT, bi, bh, wnp):
    h_new, m_split = pl.pallas_call(
        _gru_mm_body,
        out_shape=(
            jax.ShapeDtypeStruct((N, D_H), jnp.float32),
            jax.ShapeDtypeStruct((2, N, DHALF), jnp.float32),
        ),
        in_specs=[
            pl.BlockSpec((BN, DHALF), lambda i: (i, 0)),
            pl.BlockSpec((BN, DHALF), lambda i: (GRID_N + i, 0)),
            pl.BlockSpec((BN, D_H), lambda i: (i, 0)),
            pl.BlockSpec((D_H, 3 * D_H), lambda i: (0, 0)),
            pl.BlockSpec((D_H, 3 * D_H), lambda i: (0, 0)),
            pl.BlockSpec((1, 3 * D_H), lambda i: (0, 0)),
            pl.BlockSpec((1, 3 * D_H), lambda i: (0, 0)),
            pl.BlockSpec((D_H, D_H), lambda i: (0, 0)),
        ],
        out_specs=(
            pl.BlockSpec((BN, D_H), lambda i: (i, 0)),
            pl.BlockSpec((2, BN, DHALF), lambda i: (0, i, 0)),
        ),
        grid=(GRID_N,),
    )(agg_flat, agg_flat, h, wiT, whT, bi, bh, wnp)
    return h_new, m_split.reshape(2 * N, DHALF)


def _gru_final_body(a0_ref, a1_ref, h_ref, wi_ref, wh_ref, bi_ref, bh_ref,
                    fw_ref, fb_ref, o_ref):
    h_new = _gru_core(a0_ref, a1_ref, h_ref, wi_ref, wh_ref, bi_ref, bh_ref)
    h_new = jnp.maximum(h_new, 0.0)
    o_ref[...] = jnp.dot(h_new, fw_ref[...], preferred_element_type=jnp.float32) + fb_ref[0, 0]


def _gru_final(agg_flat, h, wiT, whT, bi, bh, fwT, fb):
    return pl.pallas_call(
        _gru_final_body,
        out_shape=jax.ShapeDtypeStruct((N, 1), jnp.float32),
        in_specs=[
            pl.BlockSpec((BN, DHALF), lambda i: (i, 0)),
            pl.BlockSpec((BN, DHALF), lambda i: (GRID_N + i, 0)),
            pl.BlockSpec((BN, D_H), lambda i: (i, 0)),
            pl.BlockSpec((D_H, 3 * D_H), lambda i: (0, 0)),
            pl.BlockSpec((D_H, 3 * D_H), lambda i: (0, 0)),
            pl.BlockSpec((1, 3 * D_H), lambda i: (0, 0)),
            pl.BlockSpec((1, 3 * D_H), lambda i: (0, 0)),
            pl.BlockSpec((D_H, 1), lambda i: (0, 0)),
            pl.BlockSpec((1, 1), lambda i: (0, 0), memory_space=pltpu.SMEM),
        ],
        out_specs=pl.BlockSpec((BN, 1), lambda i: (i, 0)),
        grid=(GRID_N,),
    )(agg_flat, agg_flat, h, wiT, whT, bi, bh, fwT, fb)


_HIMASK = jnp.int32(-65536)


def _sc_agg_body(m_hbm, src_hbm, dst_hbm, attr_hbm, zeros_hbm, out_hbm,
                 srcb, dstb, attrb, gbuf0, gbuf1, aggsh, sem0, sem1):
    c = lax.axis_index("c")
    s = lax.axis_index("s")
    pltpu.sync_copy(zeros_hbm, aggsh.at[pl.ds(s * RPS, RPS)])
    plsc.subcore_barrier()

    def scale_and_scatter(gbuf, k):
        def group(g, carry3):
            av = attrb[k, pl.ds(g * 16, 16)]
            e0 = g * 16
            for el in range(16):
                sval = av[el]
                e = e0 + el
                for v in range(8):
                    gbuf[e, pl.ds(v * 16, 16)] = gbuf[e, pl.ds(v * 16, 16)] * sval
            return carry3

        lax.fori_loop(0, C // 16, group, 0)
        pltpu.sync_copy(gbuf, aggsh.at[dstb.at[k]], add=True)

    def block(b, carry):
        row0 = s * CPT + b * MB
        pltpu.sync_copy(src_hbm.at[pl.ds((c * NSUB + s) * CPT + b * MB, MB)], srcb)
        pltpu.sync_copy(dst_hbm.at[pl.ds(row0, MB)], dstb)
        pltpu.sync_copy(attr_hbm.at[pl.ds(row0, MB)], attrb)
        pltpu.make_async_copy(m_hbm.at[srcb.at[0]], gbuf0, sem0).start()

        def pair(k2, carry2):
            k = 2 * k2
            pltpu.make_async_copy(m_hbm.at[srcb.at[k + 1]], gbuf1, sem1).start()
            pltpu.make_async_copy(m_hbm.at[pl.ds(0, C)], gbuf0, sem0).wait()
            scale_and_scatter(gbuf0, k)

            @pl.when(k2 < MB // 2 - 1)
            def _():
                pltpu.make_async_copy(m_hbm.at[srcb.at[k + 2]], gbuf0, sem0).start()

            pltpu.make_async_copy(m_hbm.at[pl.ds(0, C)], gbuf1, sem1).wait()
            scale_and_scatter(gbuf1, k + 1)
            return carry2

        lax.fori_loop(0, MB // 2, pair, 0)
        return carry

    lax.fori_loop(0, NBLK, block, 0)
    plsc.subcore_barrier()
    pltpu.sync_copy(aggsh.at[pl.ds(s * RPS, RPS)],
                    out_hbm.at[pl.ds(c * NPAD + s * RPS, RPS)])


@functools.cache
def _sc_agg():
    return pl.kernel(
        _sc_agg_body,
        out_type=jax.ShapeDtypeStruct((2 * NPAD, DHALF), jnp.float32),
        mesh=plsc.VectorSubcoreMesh(core_axis_name="c", subcore_axis_name="s",
                                    num_cores=2, num_subcores=NSUB),
        scratch_types=[
            pltpu.VMEM((MB, C), jnp.int32),
            pltpu.VMEM((MB, C), jnp.int32),
            pltpu.VMEM((MB, C), jnp.float32),
            pltpu.VMEM((C, DHALF), jnp.float32),
            pltpu.VMEM((C, DHALF), jnp.float32),
            pltpu.VMEM_SHARED((NPAD, DHALF), jnp.float32),
            pltpu.SemaphoreType.DMA,
            pltpu.SemaphoreType.DMA,
        ],
    )


def kernel(x, edge_index, edge_attr, W, w_ih, w_hh, b_ih, b_hh, fc_w, fc_b):
    h = jnp.concatenate([x, jnp.zeros((N, D_H - D_IN), dtype=x.dtype)], axis=1)
    src = edge_index[0].astype(jnp.int32)
    dst = edge_index[1].astype(jnp.int32)
    attr = edge_attr.astype(jnp.float32)
    pad = EPAD - E
    src = jnp.concatenate([src, jnp.zeros((pad,), jnp.int32)])
    dst2 = jnp.concatenate([dst, jnp.zeros((pad,), jnp.int32)]).reshape(NSUB * CPT, C)
    attr2 = jnp.concatenate([attr, jnp.zeros((pad,), jnp.float32)]).reshape(NSUB * CPT, C)
    src2 = jnp.concatenate([src, src + N]).reshape(2 * NSUB * CPT, C)
    zeros = jnp.zeros((RPS, DHALF), jnp.float32)

    wiT = w_ih.T
    whT = w_hh.T
    bi = b_ih.reshape(1, 3 * D_H)
    bh = b_hh.reshape(1, 3 * D_H)
    fwT = fc_w.T
    fb = fc_b.reshape(1, 1)

    m_flat = _mm_first(h, W[0])
    for i in range(L):
        agg_pad = _sc_agg()(m_flat, src2, dst2, attr2, zeros)
        agg_flat = jnp.concatenate([agg_pad[:N], agg_pad[NPAD:NPAD + N]], axis=0)
        if i < L - 1:
            h, m_flat = _gru_mm(agg_flat, h, wiT, whT, bi, bh, W[i + 1])
        else:
            out = _gru_final(agg_flat, h, wiT, whT, bi, bh, fwT, fb)
    return out
